# baseline (speedup 1.0000x reference)
import functools

import jax
import jax.numpy as jnp
from jax import lax
from jax.experimental import pallas as pl
from jax.experimental.pallas import tpu as pltpu
from jax.experimental.pallas import tpu_sc as plsc


def _build_gather(V4, L, B):
    info = plsc.get_sparse_core_info()
    NC, NS = info.num_cores, info.num_subcores
    NW = NC * NS
    assert B % (8 * NW) == 0
    b_per_w = B // NW
    CH = 128
    mesh = plsc.VectorSubcoreMesh(core_axis_name="c", subcore_axis_name="s")

    @functools.partial(
        pl.kernel,
        mesh=mesh,
        out_type=jax.ShapeDtypeStruct((B, L), jnp.int32),
        scratch_types=[
            pltpu.VMEM((b_per_w,), jnp.int32),
            pltpu.VMEM((b_per_w, L), jnp.int32),
            pltpu.SemaphoreType.DMA,
        ],
    )
    def gather_k(table_hbm, idx4_hbm, out_hbm, idx_v, lines_v, sem):
        wid = lax.axis_index("s") * NC + lax.axis_index("c")
        base = wid * b_per_w
        pltpu.sync_copy(idx4_hbm.at[pl.ds(base, b_per_w)], idx_v)
        copies = []
        for c in range(b_per_w // CH):
            copies.append(pltpu.async_copy(
                table_hbm.at[idx_v.at[pl.ds(c * CH, CH)]],
                lines_v.at[pl.ds(c * CH, CH)], sem))
        for c in copies:
            c.wait()
        pltpu.sync_copy(lines_v, out_hbm.at[pl.ds(base, b_per_w)])

    return gather_k


def _expand(w):
    lo = lax.bitcast_convert_type(w << 16, jnp.float32)
    hi = lax.bitcast_convert_type(w & jnp.int32(-65536), jnp.float32)
    return lo, hi


def _mlp_body(q_ref, m1_ref, m0_ref, w1_ref, b1_ref, w2_ref, b2_ref, o_ref):
    q = q_ref[...]
    m1 = m1_ref[...]
    m0 = m0_ref[...]
    s1 = q[:, :64] * (1 - m1) + q[:, 64:] * m1
    s0 = s1[:, :32] * (1 - m0) + s1[:, 32:] * m0
    lo, hi = _expand(s0)
    h = jnp.concatenate([lo, hi], axis=1)
    y = lax.dot_general(h, w1_ref[...], (((1,), (1,)), ((), ())),
                        preferred_element_type=jnp.float32)
    y = jnp.maximum(y + b1_ref[...], 0.0)
    z = lax.dot_general(w2_ref[...], y, (((1,), (1,)), ((), ())),
                        preferred_element_type=jnp.float32)
    o_ref[...] = z + b2_ref[0, 0]


def kernel(x, emb, W1, b1, W2, b2):
    V, D = emb.shape
    (B,) = x.shape
    idx = x.astype(jnp.int32)
    idx4 = idx >> 2
    m1 = jnp.broadcast_to(((idx >> 1) & 1)[:, None], (B, D)).astype(jnp.int32)
    m0 = jnp.broadcast_to((idx & 1)[:, None], (B, D // 2)).astype(jnp.int32)

    bits = lax.bitcast_convert_type(emb.astype(jnp.bfloat16), jnp.uint16)
    words = (bits[:, 0::2].astype(jnp.uint32)
             | (bits[:, 1::2].astype(jnp.uint32) << 16))
    packed = words.astype(jnp.int32).reshape(V // 4, 2 * D)

    lines = _build_gather(V // 4, 2 * D, B)(packed, idx4)

    perm = jnp.concatenate([jnp.arange(0, D, 2), jnp.arange(1, D, 2)])
    W1p = W1[:, perm]

    BLK = 2048
    out = pl.pallas_call(
        _mlp_body,
        grid=(B // BLK,),
        in_specs=[
            pl.BlockSpec((BLK, 2 * D), lambda i: (i, 0)),
            pl.BlockSpec((BLK, D), lambda i: (i, 0)),
            pl.BlockSpec((BLK, D // 2), lambda i: (i, 0)),
            pl.BlockSpec((D, D), lambda i: (0, 0)),
            pl.BlockSpec((1, D), lambda i: (0, 0)),
            pl.BlockSpec((1, D), lambda i: (0, 0)),
            pl.BlockSpec((1, 1), lambda i: (0, 0)),
        ],
        out_specs=pl.BlockSpec((1, BLK), lambda i: (0, i)),
        out_shape=jax.ShapeDtypeStruct((1, B), jnp.float32),
    )(lines, m1, m0, W1p, b1.reshape(1, D), W2, b2.reshape(1, 1))
    return out.reshape(B, 1)


# speedup vs baseline: 2.9681x; 2.9681x over previous
import functools

import jax
import jax.numpy as jnp
from jax import lax
from jax.experimental import pallas as pl
from jax.experimental.pallas import tpu as pltpu
from jax.experimental.pallas import tpu_sc as plsc


def _build_gather(VT, L, B):
    info = plsc.get_sparse_core_info()
    NC, NS = info.num_cores, info.num_subcores
    NW = NC * NS
    assert B % (8 * NW) == 0
    b_per_w = B // NW
    mesh = plsc.VectorSubcoreMesh(core_axis_name="c", subcore_axis_name="s")

    @functools.partial(
        pl.kernel,
        mesh=mesh,
        out_type=jax.ShapeDtypeStruct((B, L), jnp.float32),
        scratch_types=[
            pltpu.SMEM((b_per_w,), jnp.int32),
            pltpu.VMEM((b_per_w,), jnp.int32),
            pltpu.VMEM((b_per_w, L), jnp.float32),
            pltpu.SemaphoreType.DMA,
        ],
    )
    def gather_k(table_hbm, idx2_hbm, out_hbm, idx_s, idx_v, lines_v, sem):
        wid = lax.axis_index("s") * NC + lax.axis_index("c")
        base = wid * b_per_w
        pltpu.sync_copy(idx2_hbm.at[pl.ds(base, b_per_w)], idx_v)

        def stage(g, carry):
            v = idx_v[pl.ds(g * 16, 16)]
            for j in range(16):
                idx_s[g * 16 + j] = v[j]
            return carry

        lax.fori_loop(0, b_per_w // 16, stage, 0)

        U = 16

        def issue(k0):
            for j in range(U):
                p = idx_s[k0 + j]
                pltpu.async_copy(table_hbm.at[p >> 3, p & 7, :],
                                 lines_v.at[k0 + j, :], sem)

        def drain(k0):
            pltpu.make_async_copy(
                out_hbm.at[pl.ds(0, U)], lines_v.at[pl.ds(k0, U)], sem).wait()

        issue(0)

        def burst(g, carry):
            k0 = g * U
            issue(k0)
            drain(k0 - U)
            return carry

        lax.fori_loop(1, b_per_w // U, burst, 0)
        drain(b_per_w - U)
        pltpu.sync_copy(lines_v, out_hbm.at[pl.ds(base, b_per_w)])

    return gather_k


def _mlp_body(q_ref, m0_ref, w1_ref, b1_ref, w2_ref, b2_ref, o_ref):
    D = w1_ref.shape[0]
    q = q_ref[...]
    m0 = m0_ref[...]
    lo = q[:, :D]
    hi = q[:, D:]
    h = lo + (hi - lo) * m0
    y = lax.dot_general(h, w1_ref[...], (((1,), (1,)), ((), ())),
                        preferred_element_type=jnp.float32)
    y = jnp.maximum(y + b1_ref[...], 0.0)
    z = lax.dot_general(w2_ref[...], y, (((1,), (1,)), ((), ())),
                        preferred_element_type=jnp.float32)
    o_ref[...] = z + b2_ref[0, 0]


def kernel(x, emb, W1, b1, W2, b2):
    V, D = emb.shape
    (B,) = x.shape
    idx = x.astype(jnp.int32)
    idx2 = idx >> 1
    m0 = jnp.broadcast_to((idx & 1).astype(jnp.float32)[:, None], (B, D))
    table3 = emb.reshape(V // 16, 8, 2 * D)

    lines = _build_gather(V // 16, 2 * D, B)(table3, idx2)

    BLK = 2048
    out = pl.pallas_call(
        _mlp_body,
        grid=(B // BLK,),
        in_specs=[
            pl.BlockSpec((BLK, 2 * D), lambda i: (i, 0)),
            pl.BlockSpec((BLK, D), lambda i: (i, 0)),
            pl.BlockSpec((D, D), lambda i: (0, 0)),
            pl.BlockSpec((1, D), lambda i: (0, 0)),
            pl.BlockSpec((1, D), lambda i: (0, 0)),
            pl.BlockSpec((1, 1), lambda i: (0, 0)),
        ],
        out_specs=pl.BlockSpec((1, BLK), lambda i: (0, i)),
        out_shape=jax.ShapeDtypeStruct((1, B), jnp.float32),
    )(lines, m0, W1, b1.reshape(1, D), W2, b2.reshape(1, 1))
    return out.reshape(B, 1)


# speedup vs baseline: 7.5912x; 2.5576x over previous
import functools

import jax
import jax.numpy as jnp
from jax import lax
from jax.experimental import pallas as pl
from jax.experimental.pallas import tpu as pltpu
from jax.experimental.pallas import tpu_sc as plsc


def _build_gather(VT, D, B):
    info = plsc.get_sparse_core_info()
    NC, NS = info.num_cores, info.num_subcores
    NW = NC * NS
    assert B % (8 * NW) == 0
    b_per_w = B // NW
    mesh = plsc.VectorSubcoreMesh(core_axis_name="c", subcore_axis_name="s")

    @functools.partial(
        pl.kernel,
        mesh=mesh,
        out_type=jax.ShapeDtypeStruct((B, D), jnp.float32),
        scratch_types=[
            pltpu.SMEM((b_per_w,), jnp.int32),
            pltpu.VMEM((b_per_w,), jnp.int32),
            pltpu.VMEM((b_per_w, D), jnp.float32),
            pltpu.SemaphoreType.DMA,
        ],
    )
    def gather_k(table_hbm, idx_hbm, out_hbm, idx_s, idx_v, rows_v, sem):
        wid = lax.axis_index("s") * NC + lax.axis_index("c")
        base = wid * b_per_w
        pltpu.sync_copy(idx_hbm.at[pl.ds(base, b_per_w)], idx_v)

        def stage(g, carry):
            v = idx_v[pl.ds(g * 16, 16)]
            for j in range(16):
                idx_s[g * 16 + j] = v[j]
            return carry

        lax.fori_loop(0, b_per_w // 16, stage, 0)

        U = 32

        def issue(k0):
            for j in range(U):
                i = idx_s[k0 + j]
                pltpu.async_copy(table_hbm.at[i >> 3, i & 7, :],
                                 rows_v.at[k0 + j, :], sem)

        def drain(k0):
            pltpu.make_async_copy(
                out_hbm.at[pl.ds(0, U)], rows_v.at[pl.ds(k0, U)], sem).wait()

        issue(0)

        def burst(g, carry):
            k0 = g * U
            issue(k0)
            drain(k0 - U)
            return carry

        lax.fori_loop(1, b_per_w // U, burst, 0)
        drain(b_per_w - U)
        pltpu.sync_copy(rows_v, out_hbm.at[pl.ds(base, b_per_w)])

    return gather_k


def _mlp_body(h_ref, w1_ref, b1_ref, w2_ref, b2_ref, o_ref):
    h = h_ref[...]
    y = lax.dot_general(h, w1_ref[...], (((1,), (1,)), ((), ())),
                        preferred_element_type=jnp.float32)
    y = jnp.maximum(y + b1_ref[...], 0.0)
    z = lax.dot_general(w2_ref[...], y, (((1,), (1,)), ((), ())),
                        preferred_element_type=jnp.float32)
    o_ref[...] = z + b2_ref[0, 0]


def kernel(x, emb, W1, b1, W2, b2):
    V, D = emb.shape
    (B,) = x.shape
    idx = x.astype(jnp.int32)
    table3 = emb.reshape(V // 8, 8, D)

    gathered = _build_gather(V // 8, D, B)(table3, idx)

    BLK = 2048
    out = pl.pallas_call(
        _mlp_body,
        grid=(B // BLK,),
        in_specs=[
            pl.BlockSpec((BLK, D), lambda i: (i, 0)),
            pl.BlockSpec((D, D), lambda i: (0, 0)),
            pl.BlockSpec((1, D), lambda i: (0, 0)),
            pl.BlockSpec((1, D), lambda i: (0, 0)),
            pl.BlockSpec((1, 1), lambda i: (0, 0)),
        ],
        out_specs=pl.BlockSpec((1, BLK), lambda i: (0, i)),
        out_shape=jax.ShapeDtypeStruct((1, B), jnp.float32),
    )(gathered, W1, b1.reshape(1, D), W2, b2.reshape(1, 1))
    return out.reshape(B, 1)


# speedup vs baseline: 7.7202x; 1.0170x over previous
import functools

import jax
import jax.numpy as jnp
from jax import lax
from jax.experimental import pallas as pl
from jax.experimental.pallas import tpu as pltpu
from jax.experimental.pallas import tpu_sc as plsc


def _build_gather(VT, D, B):
    info = plsc.get_sparse_core_info()
    NC, NS = info.num_cores, info.num_subcores
    NW = NC * NS
    assert B % (8 * NW) == 0
    b_per_w = B // NW
    mesh = plsc.VectorSubcoreMesh(core_axis_name="c", subcore_axis_name="s")

    @functools.partial(
        pl.kernel,
        mesh=mesh,
        out_type=jax.ShapeDtypeStruct((B, D), jnp.float32),
        scratch_types=[
            pltpu.VMEM((b_per_w,), jnp.int32),
            pltpu.VMEM((b_per_w, D), jnp.float32),
            pltpu.SemaphoreType.DMA,
        ],
    )
    def gather_k(table_hbm, idx_hbm, out_hbm, idx_v, rows_v, sem):
        wid = lax.axis_index("s") * NC + lax.axis_index("c")
        base = wid * b_per_w
        pltpu.sync_copy(idx_hbm.at[pl.ds(base, b_per_w)], idx_v)

        U = 32

        def issue(k0):
            for j0 in range(0, U, 16):
                v = idx_v[pl.ds(k0 + j0, 16)]
                for j in range(16):
                    i = v[j]
                    pltpu.async_copy(table_hbm.at[i >> 3, i & 7, :],
                                     rows_v.at[k0 + j0 + j, :], sem)

        def drain(k0):
            pltpu.make_async_copy(
                out_hbm.at[pl.ds(0, U)], rows_v.at[pl.ds(k0, U)], sem).wait()

        issue(0)

        def burst(g, carry):
            k0 = g * U
            issue(k0)
            drain(k0 - U)
            return carry

        lax.fori_loop(1, b_per_w // U, burst, 0)
        drain(b_per_w - U)
        pltpu.sync_copy(rows_v, out_hbm.at[pl.ds(base, b_per_w)])

    return gather_k


def _mlp_body(h_ref, w1_ref, b1_ref, w2_ref, b2_ref, o_ref):
    h = h_ref[...]
    y = lax.dot_general(h, w1_ref[...], (((1,), (1,)), ((), ())),
                        preferred_element_type=jnp.float32)
    y = jnp.maximum(y + b1_ref[...], 0.0)
    z = lax.dot_general(w2_ref[...], y, (((1,), (1,)), ((), ())),
                        preferred_element_type=jnp.float32)
    o_ref[...] = z + b2_ref[0, 0]


def kernel(x, emb, W1, b1, W2, b2):
    V, D = emb.shape
    (B,) = x.shape
    idx = x.astype(jnp.int32)
    table3 = emb.reshape(V // 8, 8, D)

    gathered = _build_gather(V // 8, D, B)(table3, idx)

    BLK = 4096
    out = pl.pallas_call(
        _mlp_body,
        grid=(B // BLK,),
        in_specs=[
            pl.BlockSpec((BLK, D), lambda i: (i, 0)),
            pl.BlockSpec((D, D), lambda i: (0, 0)),
            pl.BlockSpec((1, D), lambda i: (0, 0)),
            pl.BlockSpec((1, D), lambda i: (0, 0)),
            pl.BlockSpec((1, 1), lambda i: (0, 0)),
        ],
        out_specs=pl.BlockSpec((1, BLK), lambda i: (0, i)),
        out_shape=jax.ShapeDtypeStruct((1, B), jnp.float32),
    )(gathered, W1, b1.reshape(1, D), W2, b2.reshape(1, 1))
    return out.reshape(B, 1)
